# Initial kernel scaffold; baseline (speedup 1.0000x reference)
import jax
import jax.numpy as jnp
from jax.experimental import pallas as pl


def kernel(x, edge_index, W_l, b_l, W_r):
    raise NotImplementedError("write your pallas kernel here")


# speedup vs baseline: 4.7743x; 4.7743x over previous
import functools

import jax
import jax.numpy as jnp
from jax import lax
from jax.experimental import pallas as pl
from jax.experimental.pallas import tpu as pltpu
from jax.experimental.pallas import tpu_sc as plsc

_N = 10000
_E = 160000
_D_IN = 256
_D_OUT = 512

_H = 128
_PW = 144
_NSUB = 16
_NCORE = 2
_EDGES_PER_SUB = _E // _NSUB
_CHUNK = 200
_NCHUNK = _EDGES_PER_SUB // _CHUNK
_NPAD = 10240
_ROWS_PER_SUB = _NPAD // _NSUB


def _sc_aggregate(xpad0, xpad1, src3, dst3, zrows):
    mesh = plsc.VectorSubcoreMesh(core_axis_name="c", subcore_axis_name="s")

    @functools.partial(
        pl.kernel,
        mesh=mesh,
        compiler_params=pltpu.CompilerParams(use_tc_tiling_on_sc=False),
        out_type=jax.ShapeDtypeStruct((_NCORE, _NPAD, _PW), jnp.float32),
        scratch_types=[
            pltpu.VMEM((_CHUNK,), jnp.int32),
            pltpu.VMEM((_CHUNK,), jnp.int32),
            pltpu.VMEM((_CHUNK, _PW), jnp.float32),
            pltpu.VMEM_SHARED((_NPAD, _PW), jnp.float32),
            pltpu.SemaphoreType.DMA,
        ],
    )
    def k(x0_hbm, x1_hbm, src_hbm, dst_hbm, z_hbm, out_hbm,
          src_v, dst_v, rows_v, acc, sem):
        c = lax.axis_index("c")
        s = lax.axis_index("s")
        row0 = s * _ROWS_PER_SUB

        pltpu.sync_copy(z_hbm, acc.at[pl.ds(row0, _ROWS_PER_SUB)])
        plsc.subcore_barrier()

        def body(j, carry):
            pltpu.sync_copy(src_hbm.at[s, j], src_v)
            pltpu.sync_copy(dst_hbm.at[s, j], dst_v)

            @pl.when(c == 0)
            def _():
                pltpu.async_copy(x0_hbm.at[src_v], rows_v, sem).wait()

            @pl.when(c == 1)
            def _():
                pltpu.async_copy(x1_hbm.at[src_v], rows_v, sem).wait()

            pltpu.sync_copy(rows_v, acc.at[dst_v], add=True)
            return carry

        lax.fori_loop(0, _NCHUNK, body, 0)
        plsc.subcore_barrier()

        pltpu.sync_copy(acc.at[pl.ds(row0, _ROWS_PER_SUB)],
                        out_hbm.at[c, pl.ds(row0, _ROWS_PER_SUB)])

    return k(xpad0, xpad1, src3, dst3, zrows)


_TC_ROWS = 1000


def _tc_linear(acc0, acc1, x, wl0, wl1, wr, b):
    def body(a0_ref, a1_ref, x_ref, wl0_ref, wl1_ref, wr_ref, b_ref, o_ref):
        a0 = a0_ref[...]
        a1 = a1_ref[...]
        cnt = a0[:, _H:_H + 1]
        denom = jnp.maximum(cnt, 1.0)
        n0 = a0[:, :_H] / denom
        n1 = a1[:, :_H] / denom
        dims = (((1,), (1,)), ((), ()))
        out = (lax.dot_general(n0, wl0_ref[...], dims,
                               preferred_element_type=jnp.float32)
               + lax.dot_general(n1, wl1_ref[...], dims,
                                 preferred_element_type=jnp.float32)
               + lax.dot_general(x_ref[...], wr_ref[...], dims,
                                 preferred_element_type=jnp.float32)
               + b_ref[...])
        o_ref[...] = jnp.maximum(out, 0.0)

    grid = (_N // _TC_ROWS,)
    return pl.pallas_call(
        body,
        grid=grid,
        in_specs=[
            pl.BlockSpec((_TC_ROWS, _PW), lambda i: (i, 0)),
            pl.BlockSpec((_TC_ROWS, _PW), lambda i: (i, 0)),
            pl.BlockSpec((_TC_ROWS, _D_IN), lambda i: (i, 0)),
            pl.BlockSpec((_D_OUT, _H), lambda i: (0, 0)),
            pl.BlockSpec((_D_OUT, _H), lambda i: (0, 0)),
            pl.BlockSpec((_D_OUT, _D_IN), lambda i: (0, 0)),
            pl.BlockSpec((1, _D_OUT), lambda i: (0, 0)),
        ],
        out_specs=pl.BlockSpec((_TC_ROWS, _D_OUT), lambda i: (i, 0)),
        out_shape=jax.ShapeDtypeStruct((_N, _D_OUT), jnp.float32),
    )(acc0, acc1, x, wl0, wl1, wr, b)


def kernel(x, edge_index, W_l, b_l, W_r):
    src = edge_index[0]
    dst = edge_index[1]

    ones = jnp.ones((_N, 1), jnp.float32)
    zpad = jnp.zeros((_N, _PW - _H - 1), jnp.float32)
    xpad0 = jnp.concatenate([x[:, :_H], ones, zpad], axis=1)
    xpad1 = jnp.concatenate([x[:, _H:], ones, zpad], axis=1)
    src3 = src.reshape(_NSUB, _NCHUNK, _CHUNK)
    dst3 = dst.reshape(_NSUB, _NCHUNK, _CHUNK)
    zrows = jnp.zeros((_ROWS_PER_SUB, _PW), jnp.float32)

    acc = _sc_aggregate(xpad0, xpad1, src3, dst3, zrows)

    return _tc_linear(acc[0, :_N], acc[1, :_N], x,
                      W_l[:, :_H], W_l[:, _H:], W_r,
                      b_l.reshape(1, _D_OUT))


# speedup vs baseline: 5.5623x; 1.1650x over previous
import functools

import jax
import jax.numpy as jnp
from jax import lax
from jax.experimental import pallas as pl
from jax.experimental.pallas import tpu as pltpu
from jax.experimental.pallas import tpu_sc as plsc

_N = 10000
_E = 160000
_D_IN = 256
_D_OUT = 512

_H = 128
_PW = 144
_NSUB = 16
_NCORE = 2
_EDGES_PER_SUB = _E // _NSUB
_CHUNK = 40
_NCHUNK = _EDGES_PER_SUB // _CHUNK
_NPAIR = _NCHUNK // 2
_NPAD = 10240
_ROWS_PER_SUB = _NPAD // _NSUB


def _sc_aggregate(xpad0, xpad1, src3, dst3, zrows):
    mesh = plsc.VectorSubcoreMesh(core_axis_name="c", subcore_axis_name="s")

    @functools.partial(
        pl.kernel,
        mesh=mesh,
        compiler_params=pltpu.CompilerParams(use_tc_tiling_on_sc=False),
        out_type=jax.ShapeDtypeStruct((_NCORE, _NPAD, _PW), jnp.float32),
        scratch_types=[
            pltpu.VMEM((_NCHUNK, _CHUNK), jnp.int32),
            pltpu.VMEM((_NCHUNK, _CHUNK), jnp.int32),
            pltpu.VMEM((_CHUNK, _PW), jnp.float32),
            pltpu.VMEM((_CHUNK, _PW), jnp.float32),
            pltpu.VMEM_SHARED((_NPAD, _PW), jnp.float32),
            pltpu.SemaphoreType.DMA,
            pltpu.SemaphoreType.DMA,
        ],
    )
    def k(x0_hbm, x1_hbm, src_hbm, dst_hbm, z_hbm, out_hbm,
          src_v, dst_v, rows0_v, rows1_v, acc, sem0, sem1):
        c = lax.axis_index("c")
        s = lax.axis_index("s")
        row0 = s * _ROWS_PER_SUB

        pltpu.sync_copy(z_hbm, acc.at[pl.ds(row0, _ROWS_PER_SUB)])
        pltpu.sync_copy(src_hbm.at[s], src_v)
        pltpu.sync_copy(dst_hbm.at[s], dst_v)
        plsc.subcore_barrier()

        def gather(j, rows_v, sem):
            @pl.when(c == 0)
            def _():
                pltpu.async_copy(x0_hbm.at[src_v.at[j]], rows_v, sem)

            @pl.when(c == 1)
            def _():
                pltpu.async_copy(x1_hbm.at[src_v.at[j]], rows_v, sem)

        def wait(rows_v, sem):
            pltpu.make_async_copy(x0_hbm.at[pl.ds(0, _CHUNK)], rows_v, sem).wait()

        gather(0, rows0_v, sem0)

        def body(i, carry):
            j = 2 * i
            gather(j + 1, rows1_v, sem1)
            wait(rows0_v, sem0)
            pltpu.sync_copy(rows0_v, acc.at[dst_v.at[j]], add=True)

            @pl.when(i < _NPAIR - 1)
            def _():
                gather(j + 2, rows0_v, sem0)

            wait(rows1_v, sem1)
            pltpu.sync_copy(rows1_v, acc.at[dst_v.at[j + 1]], add=True)
            return carry

        lax.fori_loop(0, _NPAIR, body, 0)
        plsc.subcore_barrier()

        pltpu.sync_copy(acc.at[pl.ds(row0, _ROWS_PER_SUB)],
                        out_hbm.at[c, pl.ds(row0, _ROWS_PER_SUB)])

    return k(xpad0, xpad1, src3, dst3, zrows)


_TC_ROWS = 1000


def _tc_linear(acc0, acc1, x, wl0, wl1, wr, b):
    def body(a0_ref, a1_ref, x_ref, wl0_ref, wl1_ref, wr_ref, b_ref, o_ref):
        a0 = a0_ref[...]
        a1 = a1_ref[...]
        cnt = a0[:, _H:_H + 1]
        denom = jnp.maximum(cnt, 1.0)
        n0 = a0[:, :_H] / denom
        n1 = a1[:, :_H] / denom
        dims = (((1,), (1,)), ((), ()))
        out = (lax.dot_general(n0, wl0_ref[...], dims,
                               preferred_element_type=jnp.float32)
               + lax.dot_general(n1, wl1_ref[...], dims,
                                 preferred_element_type=jnp.float32)
               + lax.dot_general(x_ref[...], wr_ref[...], dims,
                                 preferred_element_type=jnp.float32)
               + b_ref[...])
        o_ref[...] = jnp.maximum(out, 0.0)

    grid = (_N // _TC_ROWS,)
    return pl.pallas_call(
        body,
        grid=grid,
        in_specs=[
            pl.BlockSpec((_TC_ROWS, _PW), lambda i: (i, 0)),
            pl.BlockSpec((_TC_ROWS, _PW), lambda i: (i, 0)),
            pl.BlockSpec((_TC_ROWS, _D_IN), lambda i: (i, 0)),
            pl.BlockSpec((_D_OUT, _H), lambda i: (0, 0)),
            pl.BlockSpec((_D_OUT, _H), lambda i: (0, 0)),
            pl.BlockSpec((_D_OUT, _D_IN), lambda i: (0, 0)),
            pl.BlockSpec((1, _D_OUT), lambda i: (0, 0)),
        ],
        out_specs=pl.BlockSpec((_TC_ROWS, _D_OUT), lambda i: (i, 0)),
        out_shape=jax.ShapeDtypeStruct((_N, _D_OUT), jnp.float32),
    )(acc0, acc1, x, wl0, wl1, wr, b)


def kernel(x, edge_index, W_l, b_l, W_r):
    src = edge_index[0]
    dst = edge_index[1]

    ones = jnp.ones((_N, 1), jnp.float32)
    zpad = jnp.zeros((_N, _PW - _H - 1), jnp.float32)
    xpad0 = jnp.concatenate([x[:, :_H], ones, zpad], axis=1)
    xpad1 = jnp.concatenate([x[:, _H:], ones, zpad], axis=1)
    src3 = src.reshape(_NSUB, _NCHUNK, _CHUNK)
    dst3 = dst.reshape(_NSUB, _NCHUNK, _CHUNK)
    zrows = jnp.zeros((_ROWS_PER_SUB, _PW), jnp.float32)

    acc = _sc_aggregate(xpad0, xpad1, src3, dst3, zrows)

    return _tc_linear(acc[0, :_N], acc[1, :_N], x,
                      W_l[:, :_H], W_l[:, _H:], W_r,
                      b_l.reshape(1, _D_OUT))


# speedup vs baseline: 5.6768x; 1.0206x over previous
import functools

import jax
import jax.numpy as jnp
from jax import lax
from jax.experimental import pallas as pl
from jax.experimental.pallas import tpu as pltpu
from jax.experimental.pallas import tpu_sc as plsc

_N = 10000
_E = 160000
_D_IN = 256
_D_OUT = 512

_H = 128
_PW = 144
_NSUB = 16
_NCORE = 2
_EDGES_PER_SUB = _E // _NSUB
_CHUNK = 40
_NCHUNK = _EDGES_PER_SUB // _CHUNK
_NPAIR = _NCHUNK // 2
_NPAD = 10240
_ROWS_PER_SUB = _NPAD // _NSUB


def _sc_aggregate(xpad0, xpad1, src3, dst3, zrows):
    mesh = plsc.VectorSubcoreMesh(core_axis_name="c", subcore_axis_name="s")

    @functools.partial(
        pl.kernel,
        mesh=mesh,
        compiler_params=pltpu.CompilerParams(use_tc_tiling_on_sc=False),
        out_type=jax.ShapeDtypeStruct((_NCORE, _NPAD, _PW), jnp.float32),
        scratch_types=[
            pltpu.VMEM((_NCHUNK, _CHUNK), jnp.int32),
            pltpu.VMEM((_NCHUNK, _CHUNK), jnp.int32),
            pltpu.VMEM((_CHUNK, _PW), jnp.float32),
            pltpu.VMEM((_CHUNK, _PW), jnp.float32),
            pltpu.VMEM_SHARED((_NPAD, _PW), jnp.float32),
            pltpu.SemaphoreType.DMA,
            pltpu.SemaphoreType.DMA,
        ],
    )
    def k(x0_hbm, x1_hbm, src_hbm, dst_hbm, z_hbm, out_hbm,
          src_v, dst_v, rows0_v, rows1_v, acc, sem0, sem1):
        c = lax.axis_index("c")
        s = lax.axis_index("s")
        row0 = s * _ROWS_PER_SUB

        pltpu.sync_copy(z_hbm, acc.at[pl.ds(row0, _ROWS_PER_SUB)])
        pltpu.sync_copy(src_hbm.at[s], src_v)
        pltpu.sync_copy(dst_hbm.at[s], dst_v)
        plsc.subcore_barrier()

        def gather(j, rows_v, sem):
            @pl.when(c == 0)
            def _():
                pltpu.async_copy(x0_hbm.at[src_v.at[j]], rows_v, sem)

            @pl.when(c == 1)
            def _():
                pltpu.async_copy(x1_hbm.at[src_v.at[j]], rows_v, sem)

        def wait(rows_v, sem):
            pltpu.make_async_copy(x0_hbm.at[pl.ds(0, _CHUNK)], rows_v, sem).wait()

        gather(0, rows0_v, sem0)

        def body(i, carry):
            j = 2 * i
            gather(j + 1, rows1_v, sem1)
            wait(rows0_v, sem0)
            pltpu.sync_copy(rows0_v, acc.at[dst_v.at[j]], add=True)

            @pl.when(i < _NPAIR - 1)
            def _():
                gather(j + 2, rows0_v, sem0)

            wait(rows1_v, sem1)
            pltpu.sync_copy(rows1_v, acc.at[dst_v.at[j + 1]], add=True)
            return carry

        lax.fori_loop(0, _NPAIR, body, 0)
        plsc.subcore_barrier()

        pltpu.sync_copy(acc.at[pl.ds(row0, _ROWS_PER_SUB)],
                        out_hbm.at[c, pl.ds(row0, _ROWS_PER_SUB)])

    return k(xpad0, xpad1, src3, dst3, zrows)


_TC_ROWS = 1000


def _tc_root(x, wr, b):
    def body(x_ref, wr_ref, b_ref, o_ref):
        dims = (((1,), (1,)), ((), ()))
        o_ref[...] = lax.dot_general(x_ref[...], wr_ref[...], dims,
                                     preferred_element_type=jnp.float32) + b_ref[...]

    return pl.pallas_call(
        body,
        grid=(_N // _TC_ROWS,),
        in_specs=[
            pl.BlockSpec((_TC_ROWS, _D_IN), lambda i: (i, 0)),
            pl.BlockSpec((_D_OUT, _D_IN), lambda i: (0, 0)),
            pl.BlockSpec((1, _D_OUT), lambda i: (0, 0)),
        ],
        out_specs=pl.BlockSpec((_TC_ROWS, _D_OUT), lambda i: (i, 0)),
        out_shape=jax.ShapeDtypeStruct((_N, _D_OUT), jnp.float32),
    )(x, wr, b)


def _tc_combine(acc, partial, wl0, wl1):
    def body(a0_ref, a1_ref, p_ref, wl0_ref, wl1_ref, o_ref):
        a0 = a0_ref[0]
        a1 = a1_ref[0]
        denom = jnp.maximum(a0[:, _H:_H + 1], 1.0)
        n0 = a0[:, :_H] / denom
        n1 = a1[:, :_H] / denom
        dims = (((1,), (1,)), ((), ()))
        out = (lax.dot_general(n0, wl0_ref[...], dims,
                               preferred_element_type=jnp.float32)
               + lax.dot_general(n1, wl1_ref[...], dims,
                                 preferred_element_type=jnp.float32)
               + p_ref[...])
        o_ref[...] = jnp.maximum(out, 0.0)

    return pl.pallas_call(
        body,
        grid=(_N // _TC_ROWS,),
        in_specs=[
            pl.BlockSpec((1, _TC_ROWS, _PW), lambda i: (0, i, 0)),
            pl.BlockSpec((1, _TC_ROWS, _PW), lambda i: (1, i, 0)),
            pl.BlockSpec((_TC_ROWS, _D_OUT), lambda i: (i, 0)),
            pl.BlockSpec((_D_OUT, _H), lambda i: (0, 0)),
            pl.BlockSpec((_D_OUT, _H), lambda i: (0, 0)),
        ],
        out_specs=pl.BlockSpec((_TC_ROWS, _D_OUT), lambda i: (i, 0)),
        out_shape=jax.ShapeDtypeStruct((_N, _D_OUT), jnp.float32),
    )(acc, acc, partial, wl0, wl1)


def kernel(x, edge_index, W_l, b_l, W_r):
    src = edge_index[0]
    dst = edge_index[1]

    ones = jnp.ones((_N, 1), jnp.float32)
    zpad = jnp.zeros((_N, _PW - _H - 1), jnp.float32)
    xpad0 = jnp.concatenate([x[:, :_H], ones, zpad], axis=1)
    xpad1 = jnp.concatenate([x[:, _H:], ones, zpad], axis=1)
    src3 = src.reshape(_NSUB, _NCHUNK, _CHUNK)
    dst3 = dst.reshape(_NSUB, _NCHUNK, _CHUNK)
    zrows = jnp.zeros((_ROWS_PER_SUB, _PW), jnp.float32)

    acc = _sc_aggregate(xpad0, xpad1, src3, dst3, zrows)
    partial = _tc_root(x, W_r, b_l.reshape(1, _D_OUT))

    return _tc_combine(acc, partial, W_l[:, :_H], W_l[:, _H:])


# speedup vs baseline: 6.5923x; 1.1613x over previous
import functools

import jax
import jax.numpy as jnp
from jax import lax
from jax.experimental import pallas as pl
from jax.experimental.pallas import tpu as pltpu
from jax.experimental.pallas import tpu_sc as plsc

_N = 10000
_E = 160000
_D_IN = 256
_D_OUT = 512

_H = 128
_PW = 144
_NSUB = 16
_NCORE = 2
_EDGES_PER_SUB = _E // _NSUB
_CHUNK = 80
_NCHUNK = _EDGES_PER_SUB // _CHUNK
_NPAD = 10240
_ROWS_PER_SUB = _NPAD // _NSUB


def _sc_aggregate(xpad0, xpad1, src3, dst3, zrows):
    mesh = plsc.VectorSubcoreMesh(core_axis_name="c", subcore_axis_name="s")

    @functools.partial(
        pl.kernel,
        mesh=mesh,
        compiler_params=pltpu.CompilerParams(use_tc_tiling_on_sc=False),
        out_type=jax.ShapeDtypeStruct((_NCORE, _NPAD, _PW), jnp.float32),
        scratch_types=[
            pltpu.VMEM((_CHUNK,), jnp.int32),
            pltpu.VMEM((_CHUNK,), jnp.int32),
            pltpu.VMEM((_NCHUNK, _CHUNK), jnp.int32),
            pltpu.VMEM((_CHUNK, _PW), jnp.float32),
            pltpu.VMEM((_CHUNK, _PW), jnp.float32),
            pltpu.VMEM_SHARED((_NPAD, _PW), jnp.float32),
            pltpu.SemaphoreType.DMA,
            pltpu.SemaphoreType.DMA,
            pltpu.SemaphoreType.DMA,
            pltpu.SemaphoreType.DMA,
        ],
    )
    def k(x0_hbm, x1_hbm, src_hbm, dst_hbm, z_hbm, out_hbm,
          srcb0, srcb1, dst_v, rows0_v, rows1_v, acc,
          rsem0, rsem1, isem0, isem1):
        c = lax.axis_index("c")
        s = lax.axis_index("s")
        row0 = s * _ROWS_PER_SUB

        srcb = (srcb0, srcb1)
        rows = (rows0_v, rows1_v)
        rsem = (rsem0, rsem1)
        isem = (isem0, isem1)

        pltpu.sync_copy(z_hbm, acc.at[pl.ds(row0, _ROWS_PER_SUB)])
        pltpu.sync_copy(dst_hbm.at[s], dst_v)
        plsc.subcore_barrier()

        def gather(p, rows_v, sem):
            @pl.when(c == 0)
            def _():
                pltpu.async_copy(x0_hbm.at[srcb[p]], rows_v, sem)

            @pl.when(c == 1)
            def _():
                pltpu.async_copy(x1_hbm.at[srcb[p]], rows_v, sem)

        def wait_rows(rows_v, sem):
            pltpu.make_async_copy(x0_hbm.at[pl.ds(0, _CHUNK)], rows_v,
                                  sem).wait()

        def wait_idx(p):
            pltpu.make_async_copy(src_hbm.at[s, 0], srcb[p], isem[p]).wait()

        pltpu.sync_copy(src_hbm.at[s, 0], srcb0)
        gather(0, rows0_v, rsem0)
        pltpu.async_copy(src_hbm.at[s, 1], srcb1, isem1)

        def step(j, p):
            @pl.when(j + 1 < _NCHUNK)
            def _():
                wait_idx(1 - p)
                gather(1 - p, rows[1 - p], rsem[1 - p])

            wait_rows(rows[p], rsem[p])

            @pl.when(j + 2 < _NCHUNK)
            def _():
                pltpu.async_copy(src_hbm.at[s, j + 2], srcb[p], isem[p])

            pltpu.sync_copy(rows[p], acc.at[dst_v.at[j]], add=True)

        def body(j, carry):
            @pl.when(j % 2 == 0)
            def _():
                step(j, 0)

            @pl.when(j % 2 == 1)
            def _():
                step(j, 1)

            return carry

        lax.fori_loop(0, _NCHUNK, body, 0)
        plsc.subcore_barrier()

        pltpu.sync_copy(acc.at[pl.ds(row0, _ROWS_PER_SUB)],
                        out_hbm.at[c, pl.ds(row0, _ROWS_PER_SUB)])

    return k(xpad0, xpad1, src3, dst3, zrows)


_TC_ROWS = 1000


def _tc_root(x, wr, b):
    def body(x_ref, wr_ref, b_ref, o_ref):
        dims = (((1,), (1,)), ((), ()))
        o_ref[...] = lax.dot_general(x_ref[...], wr_ref[...], dims,
                                     preferred_element_type=jnp.float32) + b_ref[...]

    return pl.pallas_call(
        body,
        grid=(_N // _TC_ROWS,),
        in_specs=[
            pl.BlockSpec((_TC_ROWS, _D_IN), lambda i: (i, 0)),
            pl.BlockSpec((_D_OUT, _D_IN), lambda i: (0, 0)),
            pl.BlockSpec((1, _D_OUT), lambda i: (0, 0)),
        ],
        out_specs=pl.BlockSpec((_TC_ROWS, _D_OUT), lambda i: (i, 0)),
        out_shape=jax.ShapeDtypeStruct((_N, _D_OUT), jnp.float32),
    )(x, wr, b)


def _tc_combine(acc, partial, wl0, wl1):
    def body(a0_ref, a1_ref, p_ref, wl0_ref, wl1_ref, o_ref):
        a0 = a0_ref[0]
        a1 = a1_ref[0]
        denom = jnp.maximum(a0[:, _H:_H + 1], 1.0)
        n0 = a0[:, :_H] / denom
        n1 = a1[:, :_H] / denom
        dims = (((1,), (1,)), ((), ()))
        out = (lax.dot_general(n0, wl0_ref[...], dims,
                               preferred_element_type=jnp.float32)
               + lax.dot_general(n1, wl1_ref[...], dims,
                                 preferred_element_type=jnp.float32)
               + p_ref[...])
        o_ref[...] = jnp.maximum(out, 0.0)

    return pl.pallas_call(
        body,
        grid=(_N // _TC_ROWS,),
        in_specs=[
            pl.BlockSpec((1, _TC_ROWS, _PW), lambda i: (0, i, 0)),
            pl.BlockSpec((1, _TC_ROWS, _PW), lambda i: (1, i, 0)),
            pl.BlockSpec((_TC_ROWS, _D_OUT), lambda i: (i, 0)),
            pl.BlockSpec((_D_OUT, _H), lambda i: (0, 0)),
            pl.BlockSpec((_D_OUT, _H), lambda i: (0, 0)),
        ],
        out_specs=pl.BlockSpec((_TC_ROWS, _D_OUT), lambda i: (i, 0)),
        out_shape=jax.ShapeDtypeStruct((_N, _D_OUT), jnp.float32),
    )(acc, acc, partial, wl0, wl1)


def kernel(x, edge_index, W_l, b_l, W_r):
    src = edge_index[0]
    dst = edge_index[1]

    ones = jnp.ones((_N, 1), jnp.float32)
    zpad = jnp.zeros((_N, _PW - _H - 1), jnp.float32)
    xpad0 = jnp.concatenate([x[:, :_H], ones, zpad], axis=1)
    xpad1 = jnp.concatenate([x[:, _H:], ones, zpad], axis=1)
    src3 = src.reshape(_NSUB, _NCHUNK, _CHUNK)
    dst3 = dst.reshape(_NSUB, _NCHUNK, _CHUNK)
    zrows = jnp.zeros((_ROWS_PER_SUB, _PW), jnp.float32)

    acc = _sc_aggregate(xpad0, xpad1, src3, dst3, zrows)
    partial = _tc_root(x, W_r, b_l.reshape(1, _D_OUT))

    return _tc_combine(acc, partial, W_l[:, :_H], W_l[:, _H:])


# speedup vs baseline: 6.7926x; 1.0304x over previous
import functools

import jax
import jax.numpy as jnp
from jax import lax
from jax.experimental import pallas as pl
from jax.experimental.pallas import tpu as pltpu
from jax.experimental.pallas import tpu_sc as plsc

_N = 10000
_E = 160000
_D_IN = 256
_D_OUT = 512

_H = 128
_PW = 144
_NSUB = 16
_NCORE = 2
_EDGES_PER_SUB = _E // _NSUB
_CHUNK = 80
_NCHUNK = _EDGES_PER_SUB // _CHUNK
_NPAD = 10240
_ROWS_PER_SUB = _NPAD // _NSUB


def _sc_aggregate(xpad0, xpad1, src3, dst3, zrows):
    mesh = plsc.VectorSubcoreMesh(core_axis_name="c", subcore_axis_name="s")

    @functools.partial(
        pl.kernel,
        mesh=mesh,
        compiler_params=pltpu.CompilerParams(use_tc_tiling_on_sc=False),
        out_type=jax.ShapeDtypeStruct((_NCORE, _NPAD, _PW), jnp.float32),
        scratch_types=[
            pltpu.VMEM((_CHUNK,), jnp.int32),
            pltpu.VMEM((_CHUNK,), jnp.int32),
            pltpu.VMEM((_NCHUNK, _CHUNK), jnp.int32),
            pltpu.VMEM((_CHUNK, _PW), jnp.float32),
            pltpu.VMEM((_CHUNK, _PW), jnp.float32),
            pltpu.VMEM_SHARED((_NPAD, _PW), jnp.float32),
            pltpu.SemaphoreType.DMA,
            pltpu.SemaphoreType.DMA,
            pltpu.SemaphoreType.DMA,
            pltpu.SemaphoreType.DMA,
        ],
    )
    def k(x0_hbm, x1_hbm, src_hbm, dst_hbm, z_hbm, out_hbm,
          srcb0, srcb1, dst_v, rows0_v, rows1_v, acc,
          rsem0, rsem1, isem0, isem1):
        c = lax.axis_index("c")
        s = lax.axis_index("s")
        row0 = s * _ROWS_PER_SUB

        srcb = (srcb0, srcb1)
        rows = (rows0_v, rows1_v)
        rsem = (rsem0, rsem1)
        isem = (isem0, isem1)

        pltpu.sync_copy(z_hbm, acc.at[pl.ds(row0, _ROWS_PER_SUB)])
        pltpu.sync_copy(dst_hbm.at[s], dst_v)
        plsc.subcore_barrier()

        def gather(p, rows_v, sem):
            @pl.when(c == 0)
            def _():
                pltpu.async_copy(x0_hbm.at[srcb[p]], rows_v, sem)

            @pl.when(c == 1)
            def _():
                pltpu.async_copy(x1_hbm.at[srcb[p]], rows_v, sem)

        def wait_rows(rows_v, sem):
            pltpu.make_async_copy(x0_hbm.at[pl.ds(0, _CHUNK)], rows_v,
                                  sem).wait()

        def wait_idx(p):
            pltpu.make_async_copy(src_hbm.at[s, 0], srcb[p], isem[p]).wait()

        pltpu.sync_copy(src_hbm.at[s, 0], srcb0)
        gather(0, rows0_v, rsem0)
        pltpu.async_copy(src_hbm.at[s, 1], srcb1, isem1)

        def step(j, p):
            @pl.when(j + 1 < _NCHUNK)
            def _():
                wait_idx(1 - p)
                gather(1 - p, rows[1 - p], rsem[1 - p])

            wait_rows(rows[p], rsem[p])

            @pl.when(j + 2 < _NCHUNK)
            def _():
                pltpu.async_copy(src_hbm.at[s, j + 2], srcb[p], isem[p])

            pltpu.sync_copy(rows[p], acc.at[dst_v.at[j]], add=True)

        def body(j, carry):
            @pl.when(j % 2 == 0)
            def _():
                step(j, 0)

            @pl.when(j % 2 == 1)
            def _():
                step(j, 1)

            return carry

        lax.fori_loop(0, _NCHUNK, body, 0)
        plsc.subcore_barrier()

        pltpu.sync_copy(acc.at[pl.ds(row0, _ROWS_PER_SUB)],
                        out_hbm.at[c, pl.ds(row0, _ROWS_PER_SUB)])

    return k(xpad0, xpad1, src3, dst3, zrows)


_TC_ROWS = 1000


def _tc_linear(acc, x, wl0, wl1, wr, b):
    def body(a0_ref, a1_ref, x_ref, wl0_ref, wl1_ref, wr_ref, b_ref, o_ref):
        a0 = a0_ref[0]
        a1 = a1_ref[0]
        denom = jnp.maximum(a0[:, _H:_H + 1], 1.0)
        n0 = a0[:, :_H] / denom
        n1 = a1[:, :_H] / denom
        dims = (((1,), (1,)), ((), ()))
        out = (lax.dot_general(n0, wl0_ref[...], dims,
                               preferred_element_type=jnp.float32)
               + lax.dot_general(n1, wl1_ref[...], dims,
                                 preferred_element_type=jnp.float32)
               + lax.dot_general(x_ref[...], wr_ref[...], dims,
                                 preferred_element_type=jnp.float32)
               + b_ref[...])
        o_ref[...] = jnp.maximum(out, 0.0)

    return pl.pallas_call(
        body,
        grid=(_N // _TC_ROWS,),
        in_specs=[
            pl.BlockSpec((1, _TC_ROWS, _PW), lambda i: (0, i, 0)),
            pl.BlockSpec((1, _TC_ROWS, _PW), lambda i: (1, i, 0)),
            pl.BlockSpec((_TC_ROWS, _D_IN), lambda i: (i, 0)),
            pl.BlockSpec((_D_OUT, _H), lambda i: (0, 0)),
            pl.BlockSpec((_D_OUT, _H), lambda i: (0, 0)),
            pl.BlockSpec((_D_OUT, _D_IN), lambda i: (0, 0)),
            pl.BlockSpec((1, _D_OUT), lambda i: (0, 0)),
        ],
        out_specs=pl.BlockSpec((_TC_ROWS, _D_OUT), lambda i: (i, 0)),
        out_shape=jax.ShapeDtypeStruct((_N, _D_OUT), jnp.float32),
    )(acc, acc, x, wl0, wl1, wr, b)


def kernel(x, edge_index, W_l, b_l, W_r):
    src = edge_index[0]
    dst = edge_index[1]

    ones = jnp.ones((_N, 1), jnp.float32)
    zpad = jnp.zeros((_N, _PW - _H - 1), jnp.float32)
    xpad0 = jnp.concatenate([x[:, :_H], ones, zpad], axis=1)
    xpad1 = jnp.concatenate([x[:, _H:], ones, zpad], axis=1)
    src3 = src.reshape(_NSUB, _NCHUNK, _CHUNK)
    dst3 = dst.reshape(_NSUB, _NCHUNK, _CHUNK)
    zrows = jnp.zeros((_ROWS_PER_SUB, _PW), jnp.float32)

    acc = _sc_aggregate(xpad0, xpad1, src3, dst3, zrows)

    return _tc_linear(acc, x, W_l[:, :_H], W_l[:, _H:], W_r,
                      b_l.reshape(1, _D_OUT))


# speedup vs baseline: 8.5356x; 1.2566x over previous
import functools

import jax
import jax.numpy as jnp
from jax import lax
from jax.experimental import pallas as pl
from jax.experimental.pallas import tpu as pltpu
from jax.experimental.pallas import tpu_sc as plsc

_N = 10000
_E = 160000
_D_IN = 256
_D_OUT = 512

_H = 128
_CW = 8
_NSUB = 16
_NCORE = 2
_EDGES_PER_SUB = _E // _NSUB
_CHUNK = 80
_NCHUNK = _EDGES_PER_SUB // _CHUNK
_NPAD = 10240
_ROWS_PER_SUB = _NPAD // _NSUB


def _sc_aggregate(x2, src0_3, src1_3, dst3, zrows, zcnt, ones_rows):
    mesh = plsc.VectorSubcoreMesh(core_axis_name="c", subcore_axis_name="s")

    @functools.partial(
        pl.kernel,
        mesh=mesh,
        compiler_params=pltpu.CompilerParams(use_tc_tiling_on_sc=False),
        out_type=(
            jax.ShapeDtypeStruct((_NCORE, _NPAD, _H), jnp.float32),
            jax.ShapeDtypeStruct((_NPAD, _CW), jnp.float32),
        ),
        scratch_types=[
            pltpu.VMEM((_CHUNK,), jnp.int32),
            pltpu.VMEM((_CHUNK,), jnp.int32),
            pltpu.VMEM((_NCHUNK, _CHUNK), jnp.int32),
            pltpu.VMEM((_CHUNK, _H), jnp.float32),
            pltpu.VMEM((_CHUNK, _H), jnp.float32),
            pltpu.VMEM((_CHUNK, _CW), jnp.float32),
            pltpu.VMEM_SHARED((_NPAD, _H), jnp.float32),
            pltpu.VMEM_SHARED((_NPAD, _CW), jnp.float32),
            pltpu.SemaphoreType.DMA,
            pltpu.SemaphoreType.DMA,
            pltpu.SemaphoreType.DMA,
            pltpu.SemaphoreType.DMA,
            pltpu.SemaphoreType.DMA,
            pltpu.SemaphoreType.DMA,
        ],
    )
    def k(x2_hbm, src0_hbm, src1_hbm, dst_hbm, z_hbm, zc_hbm, ones_hbm,
          feat_out, cnt_out,
          srcb0, srcb1, dst_v, rows0_v, rows1_v, ones_v, accf, accc,
          rsem0, rsem1, isem0, isem1, csem0, csem1):
        c = lax.axis_index("c")
        s = lax.axis_index("s")
        row0 = s * _ROWS_PER_SUB

        srcb = (srcb0, srcb1)
        rows = (rows0_v, rows1_v)
        rsem = (rsem0, rsem1)
        isem = (isem0, isem1)
        csem = (csem0, csem1)

        pltpu.sync_copy(z_hbm, accf.at[pl.ds(row0, _ROWS_PER_SUB)])
        pltpu.sync_copy(zc_hbm, accc.at[pl.ds(row0, _ROWS_PER_SUB)])
        pltpu.sync_copy(dst_hbm.at[s], dst_v)
        pltpu.sync_copy(ones_hbm, ones_v)
        plsc.subcore_barrier()

        def gather(p, sem):
            pltpu.async_copy(x2_hbm.at[srcb[p]], rows[p], sem)

        def load_idx(p, j):
            @pl.when(c == 0)
            def _():
                pltpu.async_copy(src0_hbm.at[s, j], srcb[p], isem[p])

            @pl.when(c == 1)
            def _():
                pltpu.async_copy(src1_hbm.at[s, j], srcb[p], isem[p])

        def wait_rows(p):
            pltpu.make_async_copy(x2_hbm.at[pl.ds(0, _CHUNK)], rows[p],
                                  rsem[p]).wait()

        def wait_idx(p):
            pltpu.make_async_copy(src0_hbm.at[s, 0], srcb[p], isem[p]).wait()

        def wait_cnt(p):
            pltpu.make_async_copy(ones_hbm, ones_v, csem[p]).wait()

        @pl.when(c == 0)
        def _():
            pltpu.sync_copy(src0_hbm.at[s, 0], srcb0)

        @pl.when(c == 1)
        def _():
            pltpu.sync_copy(src1_hbm.at[s, 0], srcb0)

        gather(0, rsem0)
        load_idx(1, 1)

        def step(j, p):
            @pl.when(j + 1 < _NCHUNK)
            def _():
                wait_idx(1 - p)
                gather(1 - p, rsem[1 - p])

            @pl.when(j >= 2)
            def _():
                wait_cnt(p)

            pltpu.async_copy(ones_v, accc.at[dst_v.at[j]], csem[p], add=True)

            wait_rows(p)

            @pl.when(j + 2 < _NCHUNK)
            def _():
                load_idx(p, j + 2)

            pltpu.sync_copy(rows[p], accf.at[dst_v.at[j]], add=True)

        def body(j, carry):
            @pl.when(j % 2 == 0)
            def _():
                step(j, 0)

            @pl.when(j % 2 == 1)
            def _():
                step(j, 1)

            return carry

        lax.fori_loop(0, _NCHUNK, body, 0)
        wait_cnt(0)
        wait_cnt(1)
        plsc.subcore_barrier()

        pltpu.sync_copy(accf.at[pl.ds(row0, _ROWS_PER_SUB)],
                        feat_out.at[c, pl.ds(row0, _ROWS_PER_SUB)])

        @pl.when(c == 0)
        def _():
            pltpu.sync_copy(accc.at[pl.ds(row0, _ROWS_PER_SUB)],
                            cnt_out.at[pl.ds(row0, _ROWS_PER_SUB)])

    return k(x2, src0_3, src1_3, dst3, zrows, zcnt, ones_rows)


_TC_ROWS = 1000


def _tc_linear(feats, cnt, x, wl0, wl1, wr, b):
    def body(a0_ref, a1_ref, c_ref, x_ref, wl0_ref, wl1_ref, wr_ref, b_ref,
             o_ref):
        denom = jnp.maximum(c_ref[:, 0:1], 1.0)
        n0 = a0_ref[0] / denom
        n1 = a1_ref[0] / denom
        dims = (((1,), (1,)), ((), ()))
        out = (lax.dot_general(n0, wl0_ref[...], dims,
                               preferred_element_type=jnp.float32)
               + lax.dot_general(n1, wl1_ref[...], dims,
                                 preferred_element_type=jnp.float32)
               + lax.dot_general(x_ref[...], wr_ref[...], dims,
                                 preferred_element_type=jnp.float32)
               + b_ref[...])
        o_ref[...] = jnp.maximum(out, 0.0)

    return pl.pallas_call(
        body,
        grid=(_N // _TC_ROWS,),
        in_specs=[
            pl.BlockSpec((1, _TC_ROWS, _H), lambda i: (0, i, 0)),
            pl.BlockSpec((1, _TC_ROWS, _H), lambda i: (1, i, 0)),
            pl.BlockSpec((_TC_ROWS, _CW), lambda i: (i, 0)),
            pl.BlockSpec((_TC_ROWS, _D_IN), lambda i: (i, 0)),
            pl.BlockSpec((_D_OUT, _H), lambda i: (0, 0)),
            pl.BlockSpec((_D_OUT, _H), lambda i: (0, 0)),
            pl.BlockSpec((_D_OUT, _D_IN), lambda i: (0, 0)),
            pl.BlockSpec((1, _D_OUT), lambda i: (0, 0)),
        ],
        out_specs=pl.BlockSpec((_TC_ROWS, _D_OUT), lambda i: (i, 0)),
        out_shape=jax.ShapeDtypeStruct((_N, _D_OUT), jnp.float32),
    )(feats, feats, cnt, x, wl0, wl1, wr, b)


def kernel(x, edge_index, W_l, b_l, W_r):
    src = edge_index[0]
    dst = edge_index[1]

    x2 = x.reshape(_NCORE * _N, _H)
    src0_3 = (2 * src).reshape(_NSUB, _NCHUNK, _CHUNK)
    src1_3 = (2 * src + 1).reshape(_NSUB, _NCHUNK, _CHUNK)
    dst3 = dst.reshape(_NSUB, _NCHUNK, _CHUNK)
    zrows = jnp.zeros((_ROWS_PER_SUB, _H), jnp.float32)
    zcnt = jnp.zeros((_ROWS_PER_SUB, _CW), jnp.float32)
    ones_rows = jnp.concatenate(
        [jnp.ones((_CHUNK, 1), jnp.float32),
         jnp.zeros((_CHUNK, _CW - 1), jnp.float32)], axis=1)

    feats, cnt = _sc_aggregate(x2, src0_3, src1_3, dst3, zrows, zcnt,
                               ones_rows)

    return _tc_linear(feats, cnt, x, W_l[:, :_H], W_l[:, _H:], W_r,
                      b_l.reshape(1, _D_OUT))


# speedup vs baseline: 8.6998x; 1.0192x over previous
import functools

import jax
import jax.numpy as jnp
from jax import lax
from jax.experimental import pallas as pl
from jax.experimental.pallas import tpu as pltpu
from jax.experimental.pallas import tpu_sc as plsc

_N = 10000
_E = 160000
_D_IN = 256
_D_OUT = 512

_H = 128
_CW = 8
_NSUB = 16
_NCORE = 2
_EDGES_PER_SUB = _E // _NSUB
_CHUNK = 80
_NCHUNK = _EDGES_PER_SUB // _CHUNK
_NPAD = 10240
_ROWS_PER_SUB = _NPAD // _NSUB


def _sc_aggregate(x2, edge3, zrows, zcnt, ones_rows):
    mesh = plsc.VectorSubcoreMesh(core_axis_name="c", subcore_axis_name="s")

    @functools.partial(
        pl.kernel,
        mesh=mesh,
        compiler_params=pltpu.CompilerParams(use_tc_tiling_on_sc=False),
        out_type=(
            jax.ShapeDtypeStruct((_NCORE, _NPAD, _H), jnp.float32),
            jax.ShapeDtypeStruct((_NPAD, _CW), jnp.float32),
        ),
        scratch_types=[
            pltpu.VMEM((_NCHUNK, _CHUNK), jnp.int32),
            pltpu.VMEM((_NCHUNK, _CHUNK), jnp.int32),
            pltpu.VMEM((_CHUNK, _H), jnp.float32),
            pltpu.VMEM((_CHUNK, _H), jnp.float32),
            pltpu.VMEM((_CHUNK, _CW), jnp.float32),
            pltpu.VMEM_SHARED((_NPAD, _H), jnp.float32),
            pltpu.VMEM_SHARED((_NPAD, _CW), jnp.float32),
            pltpu.SemaphoreType.DMA,
            pltpu.SemaphoreType.DMA,
            pltpu.SemaphoreType.DMA,
            pltpu.SemaphoreType.DMA,
        ],
    )
    def k(x2_hbm, edge_hbm, z_hbm, zc_hbm, ones_hbm,
          feat_out, cnt_out,
          src_v, dst_v, rows0_v, rows1_v, ones_v, accf, accc,
          rsem0, rsem1, csem0, csem1):
        c = lax.axis_index("c")
        s = lax.axis_index("s")
        row0 = s * _ROWS_PER_SUB

        rows = (rows0_v, rows1_v)
        rsem = (rsem0, rsem1)
        csem = (csem0, csem1)

        pltpu.sync_copy(z_hbm, accf.at[pl.ds(row0, _ROWS_PER_SUB)])
        pltpu.sync_copy(zc_hbm, accc.at[pl.ds(row0, _ROWS_PER_SUB)])
        pltpu.sync_copy(edge_hbm.at[0, s], src_v)
        pltpu.sync_copy(edge_hbm.at[1, s], dst_v)
        pltpu.sync_copy(ones_hbm, ones_v)

        def xform(j, carry):
            for t in range(_CHUNK // 16):
                v = src_v[j, pl.ds(t * 16, 16)]
                src_v[j, pl.ds(t * 16, 16)] = v + v + c
            return carry

        lax.fori_loop(0, _NCHUNK, xform, 0)
        plsc.subcore_barrier()

        def gather(j, p):
            pltpu.async_copy(x2_hbm.at[src_v.at[j]], rows[p], rsem[p])

        def wait_rows(p):
            pltpu.make_async_copy(x2_hbm.at[pl.ds(0, _CHUNK)], rows[p],
                                  rsem[p]).wait()

        def wait_cnt(p):
            pltpu.make_async_copy(ones_hbm, ones_v, csem[p]).wait()

        gather(0, 0)

        def step(j, p):
            @pl.when(j + 1 < _NCHUNK)
            def _():
                gather(j + 1, 1 - p)

            @pl.when(j >= 2)
            def _():
                wait_cnt(p)

            pltpu.async_copy(ones_v, accc.at[dst_v.at[j]], csem[p], add=True)

            wait_rows(p)
            pltpu.sync_copy(rows[p], accf.at[dst_v.at[j]], add=True)

        def body(j, carry):
            @pl.when(j % 2 == 0)
            def _():
                step(j, 0)

            @pl.when(j % 2 == 1)
            def _():
                step(j, 1)

            return carry

        lax.fori_loop(0, _NCHUNK, body, 0)
        wait_cnt(0)
        wait_cnt(1)
        plsc.subcore_barrier()

        pltpu.sync_copy(accf.at[pl.ds(row0, _ROWS_PER_SUB)],
                        feat_out.at[c, pl.ds(row0, _ROWS_PER_SUB)])

        @pl.when(c == 0)
        def _():
            pltpu.sync_copy(accc.at[pl.ds(row0, _ROWS_PER_SUB)],
                            cnt_out.at[pl.ds(row0, _ROWS_PER_SUB)])

    return k(x2, edge3, zrows, zcnt, ones_rows)


_TC_ROWS = 1000


def _tc_linear(feats, cnt, x, wl0, wl1, wr, b):
    def body(a0_ref, a1_ref, c_ref, x_ref, wl0_ref, wl1_ref, wr_ref, b_ref,
             o_ref):
        denom = jnp.maximum(c_ref[:, 0:1], 1.0)
        n0 = a0_ref[0] / denom
        n1 = a1_ref[0] / denom
        dims = (((1,), (1,)), ((), ()))
        out = (lax.dot_general(n0, wl0_ref[...], dims,
                               preferred_element_type=jnp.float32)
               + lax.dot_general(n1, wl1_ref[...], dims,
                                 preferred_element_type=jnp.float32)
               + lax.dot_general(x_ref[...], wr_ref[...], dims,
                                 preferred_element_type=jnp.float32)
               + b_ref[...])
        o_ref[...] = jnp.maximum(out, 0.0)

    return pl.pallas_call(
        body,
        grid=(_N // _TC_ROWS,),
        in_specs=[
            pl.BlockSpec((1, _TC_ROWS, _H), lambda i: (0, i, 0)),
            pl.BlockSpec((1, _TC_ROWS, _H), lambda i: (1, i, 0)),
            pl.BlockSpec((_TC_ROWS, _CW), lambda i: (i, 0)),
            pl.BlockSpec((_TC_ROWS, _D_IN), lambda i: (i, 0)),
            pl.BlockSpec((_D_OUT, _H), lambda i: (0, 0)),
            pl.BlockSpec((_D_OUT, _H), lambda i: (0, 0)),
            pl.BlockSpec((_D_OUT, _D_IN), lambda i: (0, 0)),
            pl.BlockSpec((1, _D_OUT), lambda i: (0, 0)),
        ],
        out_specs=pl.BlockSpec((_TC_ROWS, _D_OUT), lambda i: (i, 0)),
        out_shape=jax.ShapeDtypeStruct((_N, _D_OUT), jnp.float32),
    )(feats, feats, cnt, x, wl0, wl1, wr, b)


def kernel(x, edge_index, W_l, b_l, W_r):
    x2 = x.reshape(_NCORE * _N, _H)
    edge3 = edge_index.reshape(2, _NSUB, _NCHUNK, _CHUNK)
    zrows = jnp.zeros((_ROWS_PER_SUB, _H), jnp.float32)
    zcnt = jnp.zeros((_ROWS_PER_SUB, _CW), jnp.float32)
    ones_rows = jnp.concatenate(
        [jnp.ones((_CHUNK, 1), jnp.float32),
         jnp.zeros((_CHUNK, _CW - 1), jnp.float32)], axis=1)

    feats, cnt = _sc_aggregate(x2, edge3, zrows, zcnt, ones_rows)

    return _tc_linear(feats, cnt, x, W_l[:, :_H], W_l[:, _H:], W_r,
                      b_l.reshape(1, _D_OUT))
